# Initial kernel scaffold; baseline (speedup 1.0000x reference)
import jax
import jax.numpy as jnp
from jax.experimental import pallas as pl


def kernel(x, edge_index, hidden_states, cell_states, Ws, Wn, b):
    raise NotImplementedError("write your pallas kernel here")


# speedup vs baseline: 2.9356x; 2.9356x over previous
import functools

import jax
import jax.numpy as jnp
from jax import lax
from jax.experimental import pallas as pl
from jax.experimental.pallas import tpu as pltpu
from jax.experimental.pallas import tpu_sc as plsc

N = 10000
E = 320000
F = 128
NUM_LAYERS = 2
SEQ_LEN = 4

NC = 2
NS = 16
CH = 128
NCH = -(-E // (NC * NS * CH))
E_PAD = NC * NS * CH * NCH
N_PAD = 10240
RPT = N_PAD // NS
ZC = RPT // CH

def _build_agg():
    mesh = plsc.VectorSubcoreMesh(core_axis_name="c", subcore_axis_name="s",
                                  num_cores=NC, num_subcores=NS)
    return functools.partial(
        pl.kernel,
        out_type=[jax.ShapeDtypeStruct((NC, N_PAD, F), jnp.float32)] * 2,
        mesh=mesh,
        scratch_types=[
            pltpu.VMEM((NCH, CH), jnp.int32),
            pltpu.VMEM((NCH, CH), jnp.int32),
            pltpu.VMEM((CH, F), jnp.float32),
            pltpu.VMEM((32, F), jnp.float32),
            pltpu.VMEM((16,), jnp.int32),
            pltpu.VMEM_SHARED((N_PAD, F), jnp.float32),
        ],
    )(_agg_body)


def _agg_body(t0, t1, src_h, dst_h, z_h, ng_h, out0, out1,
         src_v, dst_v, buf, zbuf, ngv, acc):
    tables = (t0, t1)
    outs = (out0, out1)
    cid = lax.axis_index("c")
    sid = lax.axis_index("s")
    pltpu.sync_copy(src_h.at[cid, sid], src_v)
    pltpu.sync_copy(dst_h.at[cid, sid], dst_v)
    pltpu.sync_copy(z_h, zbuf)
    pltpu.sync_copy(ng_h, ngv)
    ng = ngv[...][0]
    row0 = sid * RPT
    for g in range(2):
        def one_gate(g=g):
            @pl.loop(0, RPT // 32)
            def _(k):
                pltpu.sync_copy(zbuf, acc.at[pl.ds(row0 + k * 32, 32)])
            plsc.subcore_barrier()

            @pl.loop(0, NCH)
            def _(j):
                pltpu.sync_copy(tables[g].at[src_v.at[j]], buf)
                pltpu.sync_copy(buf, acc.at[dst_v.at[j]], add=True)
            plsc.subcore_barrier()

            @pl.loop(0, ZC)
            def _(k):
                pltpu.sync_copy(acc.at[pl.ds(row0 + k * CH, CH)], buf)
                pltpu.sync_copy(buf, outs[g].at[cid, pl.ds(row0 + k * CH, CH)])
        if g == 0:
            one_gate()
        else:
            pl.when(ng >= 2)(one_gate)

DW = 16


_SC_CACHE = {}


def _agg(*args):
    if "agg" not in _SC_CACHE:
        _SC_CACHE["agg"] = _build_agg()
    return _SC_CACHE["agg"](*args)


BN = 512
_GRID = (-(-N_PAD // BN),)


def _rows(i):
    return (i, 0)


def _rep(i):
    return (0, 0)


def _deginv_tc(deg_parts):
    def body(d0, d1, o):
        deg = d0[:, :1] + d1[:, :1]
        o[...] = 1.0 / jnp.maximum(deg, 1.0)
    return pl.pallas_call(
        body, grid=_GRID,
        in_specs=[pl.BlockSpec((BN, F), _rows), pl.BlockSpec((BN, F), _rows)],
        out_specs=pl.BlockSpec((BN, 1), _rows),
        out_shape=jax.ShapeDtypeStruct((N, 1), jnp.float32),
    )(deg_parts[0], deg_parts[1])


def _k_pre(inp, h, wt, wb, bias):
    def body(i_ref, h_ref, wt_ref, wb_ref, b_ref, s_ref, pr_ref, pz_ref):
        acc = jnp.dot(i_ref[...], wt_ref[...], preferred_element_type=jnp.float32)
        acc += jnp.dot(h_ref[...], wb_ref[...], preferred_element_type=jnp.float32)
        acc += b_ref[...]
        s_ref[...] = acc[:, :256]
        pr_ref[...] = acc[:, 256:384]
        pz_ref[...] = acc[:, 384:512]
    return pl.pallas_call(
        body, grid=_GRID,
        in_specs=[pl.BlockSpec((BN, F), _rows), pl.BlockSpec((BN, F), _rows),
                  pl.BlockSpec((F, 512), _rep), pl.BlockSpec((F, 512), _rep),
                  pl.BlockSpec((1, 512), _rep)],
        out_specs=[pl.BlockSpec((BN, 256), _rows), pl.BlockSpec((BN, F), _rows),
                   pl.BlockSpec((BN, F), _rows)],
        out_shape=[jax.ShapeDtypeStruct((N, 256), jnp.float32),
                   jax.ShapeDtypeStruct((N, F), jnp.float32),
                   jax.ShapeDtypeStruct((N, F), jnp.float32)],
    )(inp, h, wt, wb, bias)


def _k_gate(S, pr, pz, dinv, inp, h, wt, wb, bias):
    def body(s_ref, pr0, pr1, pz0, pz1, di_ref, i_ref, h_ref, wt_ref, wb_ref,
             b_ref, r_ref, z_ref, sc_ref, pc_ref):
        di = di_ref[...]
        r = jax.nn.sigmoid(s_ref[:, :128] + (pr0[...] + pr1[...]) * di)
        z = jax.nn.sigmoid(s_ref[:, 128:] + (pz0[...] + pz1[...]) * di)
        rh = r * h_ref[...]
        acc = jnp.dot(i_ref[...], wt_ref[...], preferred_element_type=jnp.float32)
        acc += jnp.dot(rh, wb_ref[...], preferred_element_type=jnp.float32)
        acc += b_ref[...]
        r_ref[...] = r
        z_ref[...] = z
        sc_ref[...] = acc[:, :128]
        pc_ref[...] = acc[:, 128:]
    return pl.pallas_call(
        body, grid=_GRID,
        in_specs=[pl.BlockSpec((BN, 256), _rows),
                  pl.BlockSpec((BN, F), _rows), pl.BlockSpec((BN, F), _rows),
                  pl.BlockSpec((BN, F), _rows), pl.BlockSpec((BN, F), _rows),
                  pl.BlockSpec((BN, 1), _rows),
                  pl.BlockSpec((BN, F), _rows), pl.BlockSpec((BN, F), _rows),
                  pl.BlockSpec((F, 256), _rep), pl.BlockSpec((F, 256), _rep),
                  pl.BlockSpec((1, 256), _rep)],
        out_specs=[pl.BlockSpec((BN, F), _rows), pl.BlockSpec((BN, F), _rows),
                   pl.BlockSpec((BN, F), _rows), pl.BlockSpec((BN, F), _rows)],
        out_shape=[jax.ShapeDtypeStruct((N, F), jnp.float32)] * 4,
    )(S, pr[0], pr[1], pz[0], pz[1], dinv, inp, h, wt, wb, bias)


def _k_post(Sc, pc, dinv, r, z, h, c):
    def body(sc_ref, p0, p1, di_ref, r_ref, z_ref, h_ref, c_ref, nh_ref, nc_ref):
        cand = jnp.tanh(sc_ref[...] + (p0[...] + p1[...]) * di_ref[...])
        r_ = r_ref[...]
        z_ = z_ref[...]
        nh_ref[...] = z_ * h_ref[...] + (1.0 - z_) * cand
        nc_ref[...] = r_ * c_ref[...] + (1.0 - r_) * cand
    return pl.pallas_call(
        body, grid=_GRID,
        in_specs=[pl.BlockSpec((BN, F), _rows),
                  pl.BlockSpec((BN, F), _rows), pl.BlockSpec((BN, F), _rows),
                  pl.BlockSpec((BN, 1), _rows),
                  pl.BlockSpec((BN, F), _rows), pl.BlockSpec((BN, F), _rows),
                  pl.BlockSpec((BN, F), _rows), pl.BlockSpec((BN, F), _rows)],
        out_specs=[pl.BlockSpec((BN, F), _rows), pl.BlockSpec((BN, F), _rows)],
        out_shape=[jax.ShapeDtypeStruct((N, F), jnp.float32)] * 2,
    )(Sc, pc[0], pc[1], dinv, r, z, h, c)


def kernel(x, edge_index, hidden_states, cell_states, Ws, Wn, b):
    src = edge_index[0].astype(jnp.int32)
    dst = edge_index[1].astype(jnp.int32)
    pad = E_PAD - E
    src_p = jnp.concatenate([src, jnp.zeros((pad,), jnp.int32)])
    dst_p = jnp.concatenate([dst, jnp.full((pad,), N, jnp.int32)])
    src_p = src_p.reshape(NC, NS, NCH, CH)
    dst_p = dst_p.reshape(NC, NS, NCH, CH)

    zblk = jnp.zeros((32, F), jnp.float32)
    ng2 = jnp.full((16,), 2, jnp.int32)
    ng1 = jnp.full((16,), 1, jnp.int32)

    ones_tbl = jnp.ones((N, F), jnp.float32)
    deg_parts, _ = _agg(ones_tbl, ones_tbl, src_p, dst_p, zblk, ng1)
    dinv = _deginv_tc(deg_parts)

    w4t, w4b, b4, w2t, w2b, b2 = [], [], [], [], [], []
    for j in range(NUM_LAYERS):
        w4 = jnp.concatenate([Ws[j, 0], Ws[j, 1], Wn[j, 0], Wn[j, 1]], axis=1)
        w4t.append(w4[:F])
        w4b.append(w4[F:])
        b4.append(jnp.concatenate([b[j, 0], b[j, 1], jnp.zeros((256,), jnp.float32)])[None, :])
        w2 = jnp.concatenate([Ws[j, 2], Wn[j, 2]], axis=1)
        w2t.append(w2[:F])
        w2b.append(w2[F:])
        b2.append(jnp.concatenate([b[j, 2], jnp.zeros((F,), jnp.float32)])[None, :])

    H = [hidden_states[j] for j in range(NUM_LAYERS)]
    C = [cell_states[j] for j in range(NUM_LAYERS)]
    for t in range(SEQ_LEN):
        inp = x[t]
        for j in range(NUM_LAYERS):
            S, Pr, Pz = _k_pre(inp, H[j], w4t[j], w4b[j], b4[j])
            apr, apz = _agg(Pr, Pz, src_p, dst_p, zblk, ng2)
            r, z, Sc, Pc = _k_gate(S, apr, apz, dinv, inp, H[j],
                                   w2t[j], w2b[j], b2[j])
            apc, _unused = _agg(Pc, Pc, src_p, dst_p, zblk, ng1)
            nh, ncell = _k_post(Sc, apc, dinv, r, z, H[j], C[j])
            H[j], C[j] = nh, ncell
            inp = nh
    return (x, jnp.stack(H, axis=0), jnp.stack(C, axis=0))
